# baseline (speedup 1.0000x reference)
import functools
import math

import jax
import jax.numpy as jnp
from jax import lax
from jax.experimental import pallas as pl
from jax.experimental.pallas import tpu as pltpu
from jax.experimental.pallas import tpu_sc as plsc

_NC = 2
_NS = 16
_NW = _NC * _NS
_L = 16

_SUP = 4096
_GB = 128
_DEPTH = 4
_SIGMA = 0.003


@functools.lru_cache(maxsize=None)
def _make_sc_kernel(B, H, W):
    D = 32
    bpw = B // _NW
    nsup = bpw // _SUP
    nblk = _SUP // _GB
    ngrp = _GB // _L
    assert B % _NW == 0 and bpw % _SUP == 0 and _SUP % _GB == 0
    assert nblk % _DEPTH == 0
    assert (H & (H - 1)) == 0 and (W & (W - 1)) == 0

    Hf, Wf = float(H), float(W)
    knorm = 1.0 / (2.0 * math.pi * _SIGMA)

    mesh = plsc.VectorSubcoreMesh(core_axis_name="c", subcore_axis_name="s")

    @functools.partial(
        pl.kernel,
        out_type=jax.ShapeDtypeStruct((B,), jnp.float32),
        mesh=mesh,
        compiler_params=pltpu.CompilerParams(
            needs_layout_passes=False, use_tc_tiling_on_sc=False),
        scratch_types=[
            pltpu.VMEM((2 * _SUP,), jnp.float32),
            pltpu.VMEM((2 * _SUP,), jnp.float32),
            pltpu.VMEM((_SUP,), jnp.float32),
            pltpu.VMEM((_SUP,), jnp.float32),
            pltpu.VMEM((_SUP,), jnp.int32),
            pltpu.VMEM((_SUP,), jnp.int32),
            [pltpu.VMEM((_GB, 128), jnp.float32) for _ in range(_DEPTH)],
            pltpu.VMEM((_SUP,), jnp.float32),
            [pltpu.SemaphoreType.DMA for _ in range(_DEPTH)],
        ],
    )
    def ndf_kernel(u_h, s_h, tab_h, out_h,
                   uv_v, sv_v, uf_v, vf_v, idx_v, sub_v, rows, o_v, sems):
        wid = lax.axis_index("c") * _NS + lax.axis_index("s")
        wbase = wid * bpw
        iota = lax.iota(jnp.int32, _L)

        def gather_start(j, rbuf, sem):
            pltpu.make_async_copy(
                tab_h.at[idx_v.at[pl.ds(j * _GB, _GB)]], rbuf, sem).start()

        def gather_wait(j, rbuf, sem):
            pltpu.make_async_copy(
                tab_h.at[idx_v.at[pl.ds(j * _GB, _GB)]], rbuf, sem).wait()

        def sup_body(sc, carry):
            base = wbase + sc * _SUP
            pltpu.sync_copy(u_h.at[pl.ds(2 * base, 2 * _SUP)], uv_v)
            pltpu.sync_copy(s_h.at[pl.ds(2 * base, 2 * _SUP)], sv_v)

            def idx_body(g, c):
                o = g * _L
                p2 = 2 * (o + iota)
                a0 = plsc.load_gather(uv_v, [p2])
                a1 = plsc.load_gather(uv_v, [p2 + 1])
                vv = (a0 * 0.5 + 0.5) * Hf
                uu = (a1 * 0.5 + 0.5) * Wf
                vi = vv.astype(jnp.int32)
                ui = uu.astype(jnp.int32)
                vf_v[pl.ds(o, _L)] = vv - vi.astype(jnp.float32)
                uf_v[pl.ds(o, _L)] = uu - ui.astype(jnp.float32)
                idx = (ui & (H - 1)) * W + (vi & (W - 1))
                idx_v[pl.ds(o, _L)] = lax.shift_right_logical(idx, 2)
                sub_v[pl.ds(o, _L)] = lax.shift_left(idx & 3, 5)
                return c
            lax.fori_loop(0, _SUP // _L, idx_body, 0)

            for d in range(_DEPTH):
                gather_start(d, rows[d], sems[d])

            def compute_block(j, rbuf):
                def grp(k, c):
                    o = j * _GB + k * _L
                    p = k * _L + iota
                    suboff = sub_v[pl.ds(o, _L)]
                    cs = [plsc.load_gather(rbuf, [p, suboff + kk])
                          for kk in range(D)]
                    uf = uf_v[pl.ds(o, _L)]
                    vf = vf_v[pl.ds(o, _L)]

                    def h4(c0, c1, c2, c3, t):
                        return ((c3 * t + c2) * t + c1) * t + c0

                    pa2 = 2 * (o + iota)
                    ns = []
                    for ch in range(2):
                        rs = [h4(*cs[ch * 16 + i * 4: ch * 16 + i * 4 + 4], vf)
                              for i in range(4)]
                        ns.append(h4(rs[0], rs[1], rs[2], rs[3], uf))
                    s0 = plsc.load_gather(sv_v, [pa2])
                    s1 = plsc.load_gather(sv_v, [pa2 + 1])
                    d0 = (ns[0] - s0) / _SIGMA
                    d1 = (ns[1] - s1) / _SIGMA
                    o_v[pl.ds(o, _L)] = knorm * jnp.exp(-0.5 * (d0 * d0 + d1 * d1))
                    return c
                lax.fori_loop(0, ngrp, grp, 0)

            def blkn(jj, c):
                j0 = jj * _DEPTH
                for par in range(_DEPTH):
                    j = j0 + par
                    gather_wait(j, rows[par], sems[par])
                    compute_block(j, rows[par])

                    @pl.when(j + _DEPTH < nblk)
                    def _():
                        gather_start(j + _DEPTH, rows[par], sems[par])
                return c
            lax.fori_loop(0, nblk // _DEPTH, blkn, 0)

            pltpu.sync_copy(o_v, out_h.at[pl.ds(base, _SUP)])
            return carry
        lax.fori_loop(0, nsup, sup_body, 0)

    return ndf_kernel


def kernel(u, s, normal_coeff):
    H, W, C = normal_coeff.shape[0], normal_coeff.shape[1], normal_coeff.shape[2]
    B = u.shape[0]
    assert C == 2
    table = normal_coeff.reshape(H * W * C * 16 // 128, 128)
    f = _make_sc_kernel(B, H, W)
    return f(u.reshape(2 * B), s.reshape(2 * B), table)


# speedup vs baseline: 13.1630x; 13.1630x over previous
import functools
import math

import jax
import jax.numpy as jnp
from jax import lax
from jax.experimental import pallas as pl
from jax.experimental.pallas import tpu as pltpu
from jax.experimental.pallas import tpu_sc as plsc

_NC = 2
_NS = 16
_NW = _NC * _NS
_L = 16

_SUP = 4096
_GB = 128
_DEPTH = 4
_SIGMA = 0.003


@functools.lru_cache(maxsize=None)
def _make_sc_kernel(B, H, W):
    D = 32
    bpw = B // _NW
    nsup = bpw // _SUP
    nblk = _SUP // _GB
    ngrp = _GB // _L
    assert B % _NW == 0 and bpw % _SUP == 0 and _SUP % _GB == 0
    assert nblk % _DEPTH == 0
    assert (H & (H - 1)) == 0 and (W & (W - 1)) == 0

    Hf, Wf = float(H), float(W)
    knorm = 1.0 / (2.0 * math.pi * _SIGMA)

    mesh = plsc.VectorSubcoreMesh(core_axis_name="c", subcore_axis_name="s")

    @functools.partial(
        pl.kernel,
        out_type=jax.ShapeDtypeStruct((B,), jnp.float32),
        mesh=mesh,
        compiler_params=pltpu.CompilerParams(
            needs_layout_passes=False, use_tc_tiling_on_sc=False),
        scratch_types=[
            pltpu.VMEM((_SUP,), jnp.float32),
            pltpu.VMEM((_SUP,), jnp.float32),
            pltpu.VMEM((_SUP,), jnp.float32),
            pltpu.VMEM((_SUP,), jnp.float32),
            pltpu.VMEM((_SUP,), jnp.float32),
            pltpu.VMEM((_SUP,), jnp.float32),
            pltpu.VMEM((_SUP,), jnp.int32),
            [pltpu.VMEM((_GB, D), jnp.float32) for _ in range(_DEPTH)],
            pltpu.VMEM((_SUP,), jnp.float32),
            [pltpu.SemaphoreType.DMA for _ in range(_DEPTH)],
        ],
    )
    def ndf_kernel(u0_h, u1_h, s0_h, s1_h, tab_h, out_h,
                   u0_v, u1_v, s0_v, s1_v, uf_v, vf_v, idx_v,
                   rows, o_v, sems):
        wid = lax.axis_index("c") * _NS + lax.axis_index("s")
        wbase = wid * bpw
        iota = lax.iota(jnp.int32, _L)

        def gather_start(j, rbuf, sem):
            pltpu.make_async_copy(
                tab_h.at[idx_v.at[pl.ds(j * _GB, _GB)]], rbuf, sem).start()

        def gather_wait(j, rbuf, sem):
            pltpu.make_async_copy(
                tab_h.at[idx_v.at[pl.ds(j * _GB, _GB)]], rbuf, sem).wait()

        def sup_body(sc, carry):
            base = wbase + sc * _SUP
            pltpu.sync_copy(u0_h.at[pl.ds(base, _SUP)], u0_v)
            pltpu.sync_copy(u1_h.at[pl.ds(base, _SUP)], u1_v)
            pltpu.sync_copy(s0_h.at[pl.ds(base, _SUP)], s0_v)
            pltpu.sync_copy(s1_h.at[pl.ds(base, _SUP)], s1_v)

            def idx_body(g, c):
                o = g * _L
                a0 = u0_v[pl.ds(o, _L)]
                a1 = u1_v[pl.ds(o, _L)]
                vv = (a0 * 0.5 + 0.5) * Hf
                uu = (a1 * 0.5 + 0.5) * Wf
                vi = vv.astype(jnp.int32)
                ui = uu.astype(jnp.int32)
                vf_v[pl.ds(o, _L)] = vv - vi.astype(jnp.float32)
                uf_v[pl.ds(o, _L)] = uu - ui.astype(jnp.float32)
                idx_v[pl.ds(o, _L)] = (ui & (H - 1)) * W + (vi & (W - 1))
                return c
            lax.fori_loop(0, _SUP // _L, idx_body, 0)

            for d in range(_DEPTH):
                gather_start(d, rows[d], sems[d])

            def compute_block(j, rbuf):
                def grp(k, c):
                    o = j * _GB + k * _L
                    p = k * _L + iota
                    cs = [plsc.load_gather(
                              rbuf, [p, jnp.full((_L,), kk, jnp.int32)])
                          for kk in range(D)]
                    uf = uf_v[pl.ds(o, _L)]
                    vf = vf_v[pl.ds(o, _L)]

                    def h4(c0, c1, c2, c3, t):
                        return ((c3 * t + c2) * t + c1) * t + c0

                    ns = []
                    for ch in range(2):
                        rs = [h4(*cs[ch * 16 + i * 4: ch * 16 + i * 4 + 4], vf)
                              for i in range(4)]
                        ns.append(h4(rs[0], rs[1], rs[2], rs[3], uf))
                    d0 = (ns[0] - s0_v[pl.ds(o, _L)]) / _SIGMA
                    d1 = (ns[1] - s1_v[pl.ds(o, _L)]) / _SIGMA
                    o_v[pl.ds(o, _L)] = knorm * jnp.exp(-0.5 * (d0 * d0 + d1 * d1))
                    return c
                lax.fori_loop(0, ngrp, grp, 0)

            def blkn(jj, c):
                j0 = jj * _DEPTH
                for par in range(_DEPTH):
                    j = j0 + par
                    gather_wait(j, rows[par], sems[par])
                    compute_block(j, rows[par])

                    @pl.when(j + _DEPTH < nblk)
                    def _():
                        gather_start(j + _DEPTH, rows[par], sems[par])
                return c
            lax.fori_loop(0, nblk // _DEPTH, blkn, 0)

            pltpu.sync_copy(o_v, out_h.at[pl.ds(base, _SUP)])
            return carry
        lax.fori_loop(0, nsup, sup_body, 0)

    return ndf_kernel


def kernel(u, s, normal_coeff):
    H, W, C = normal_coeff.shape[0], normal_coeff.shape[1], normal_coeff.shape[2]
    B = u.shape[0]
    assert C == 2
    table = normal_coeff.reshape(H * W, C * 16)
    f = _make_sc_kernel(B, H, W)
    return f(u[:, 0], u[:, 1], s[:, 0], s[:, 1], table)


# speedup vs baseline: 13.4070x; 1.0185x over previous
import functools
import math

import jax
import jax.numpy as jnp
from jax import lax
from jax.experimental import pallas as pl
from jax.experimental.pallas import tpu as pltpu
from jax.experimental.pallas import tpu_sc as plsc

_NC = 2
_NS = 16
_NW = _NC * _NS
_L = 16

_SUP = 4096
_GB = 128
_DEPTH = 4
_SIGMA = 0.003


@functools.lru_cache(maxsize=None)
def _make_sc_kernel(B, H, W):
    D = 32
    bpw = B // _NW
    nsup = bpw // _SUP
    nblk = _SUP // _GB
    ngrp = _GB // _L
    assert B % _NW == 0 and bpw % _SUP == 0 and _SUP % _GB == 0
    assert nblk % _DEPTH == 0
    assert (H & (H - 1)) == 0 and (W & (W - 1)) == 0

    Hf, Wf = float(H), float(W)
    knorm = 1.0 / (2.0 * math.pi * _SIGMA)

    mesh = plsc.VectorSubcoreMesh(core_axis_name="c", subcore_axis_name="s")

    @functools.partial(
        pl.kernel,
        out_type=jax.ShapeDtypeStruct((B,), jnp.float32),
        mesh=mesh,
        compiler_params=pltpu.CompilerParams(
            needs_layout_passes=False, use_tc_tiling_on_sc=False),
        scratch_types=[
            pltpu.VMEM((_SUP,), jnp.float32),
            pltpu.VMEM((_SUP,), jnp.float32),
            pltpu.VMEM((_SUP,), jnp.float32),
            pltpu.VMEM((_SUP,), jnp.float32),
            pltpu.VMEM((_SUP,), jnp.float32),
            pltpu.VMEM((_SUP,), jnp.float32),
            pltpu.VMEM((_SUP,), jnp.int32),
            [pltpu.VMEM((_GB, D), jnp.float32) for _ in range(_DEPTH)],
            pltpu.VMEM((_SUP,), jnp.float32),
            [pltpu.SemaphoreType.DMA for _ in range(_DEPTH)],
        ],
    )
    def ndf_kernel(u0_h, u1_h, s0_h, s1_h, tab_h, out_h,
                   u0_v, u1_v, s0_v, s1_v, uf_v, vf_v, idx_v,
                   rows, o_v, sems):
        wid = lax.axis_index("c") * _NS + lax.axis_index("s")
        wbase = wid * bpw
        iota = lax.iota(jnp.int32, _L)

        def gather_start(j, rbuf, sem):
            pltpu.make_async_copy(
                tab_h.at[idx_v.at[pl.ds(j * _GB, _GB)]], rbuf, sem).start()

        def gather_wait(j, rbuf, sem):
            pltpu.make_async_copy(
                tab_h.at[idx_v.at[pl.ds(j * _GB, _GB)]], rbuf, sem).wait()

        def sup_body(sc, carry):
            base = wbase + sc * _SUP
            pltpu.sync_copy(u0_h.at[pl.ds(base, _SUP)], u0_v)
            pltpu.sync_copy(u1_h.at[pl.ds(base, _SUP)], u1_v)
            pltpu.sync_copy(s0_h.at[pl.ds(base, _SUP)], s0_v)
            pltpu.sync_copy(s1_h.at[pl.ds(base, _SUP)], s1_v)

            @plsc.parallel_loop(0, _SUP // _L, unroll=4)
            def idx_body(g):
                o = g * _L
                a0 = u0_v[pl.ds(o, _L)]
                a1 = u1_v[pl.ds(o, _L)]
                vv = (a0 * 0.5 + 0.5) * Hf
                uu = (a1 * 0.5 + 0.5) * Wf
                vi = vv.astype(jnp.int32)
                ui = uu.astype(jnp.int32)
                vf_v[pl.ds(o, _L)] = vv - vi.astype(jnp.float32)
                uf_v[pl.ds(o, _L)] = uu - ui.astype(jnp.float32)
                idx_v[pl.ds(o, _L)] = (ui & (H - 1)) * W + (vi & (W - 1))

            for d in range(_DEPTH):
                gather_start(d, rows[d], sems[d])

            def compute_block(j, rbuf):
                @plsc.parallel_loop(0, ngrp, unroll=2)
                def grp(k):
                    o = j * _GB + k * _L
                    p = k * _L + iota
                    cs = [plsc.load_gather(
                              rbuf, [p, jnp.full((_L,), kk, jnp.int32)])
                          for kk in range(D)]
                    uf = uf_v[pl.ds(o, _L)]
                    vf = vf_v[pl.ds(o, _L)]

                    def h4(c0, c1, c2, c3, t):
                        return ((c3 * t + c2) * t + c1) * t + c0

                    ns = []
                    for ch in range(2):
                        rs = [h4(*cs[ch * 16 + i * 4: ch * 16 + i * 4 + 4], vf)
                              for i in range(4)]
                        ns.append(h4(rs[0], rs[1], rs[2], rs[3], uf))
                    d0 = (ns[0] - s0_v[pl.ds(o, _L)]) / _SIGMA
                    d1 = (ns[1] - s1_v[pl.ds(o, _L)]) / _SIGMA
                    o_v[pl.ds(o, _L)] = knorm * jnp.exp(-0.5 * (d0 * d0 + d1 * d1))

            def blkn(jj, c):
                j0 = jj * _DEPTH
                for par in range(_DEPTH):
                    j = j0 + par
                    gather_wait(j, rows[par], sems[par])
                    compute_block(j, rows[par])

                    @pl.when(j + _DEPTH < nblk)
                    def _():
                        gather_start(j + _DEPTH, rows[par], sems[par])
                return c
            lax.fori_loop(0, nblk // _DEPTH, blkn, 0)

            pltpu.sync_copy(o_v, out_h.at[pl.ds(base, _SUP)])
            return carry
        lax.fori_loop(0, nsup, sup_body, 0)

    return ndf_kernel


def kernel(u, s, normal_coeff):
    H, W, C = normal_coeff.shape[0], normal_coeff.shape[1], normal_coeff.shape[2]
    B = u.shape[0]
    assert C == 2
    table = normal_coeff.reshape(H * W, C * 16)
    f = _make_sc_kernel(B, H, W)
    return f(u[:, 0], u[:, 1], s[:, 0], s[:, 1], table)
